# speedup vs baseline: 1.4157x; 1.4157x over previous
import jax
import jax.numpy as jnp
from jax.experimental import pallas as pl
from jax.experimental.pallas import tpu as pltpu

_N = 50000
_NFEAT = 8
_DIMS = 128
_VS = 2.0 / 128.0
_VMIN = -1.0
_K = 5
_EPS = 1e-06


def _norm_body(dens_ref, feats_ref, out_ref):
    d = dens_ref[...]
    f = feats_ref[...]
    r = 1.0 / jnp.maximum(d, _EPS)
    out_ref[...] = jnp.concatenate([d, f * r], axis=1)


def kernel(means3d, covs, opacities, features):
    n_dims = features.shape[1]
    variances = jnp.diagonal(covs, axis1=-2, axis2=-1)
    inv_var = 1.0 / variances
    radii = 3.0 * jnp.sqrt(variances)
    start = jnp.maximum(0, ((means3d - radii - _VMIN) / _VS).astype(jnp.int32))
    end = jnp.minimum(_DIMS, ((means3d + radii - _VMIN) / _VS).astype(jnp.int32) + 1)
    offs = jnp.arange(_K, dtype=jnp.int32)
    ox, oy, oz = jnp.meshgrid(offs, offs, offs, indexing="ij")
    offsets = jnp.stack([ox, oy, oz], axis=-1).reshape(-1, 3)
    vox = start[:, None, :] + offsets[None, :, :]
    valid = jnp.all(vox < end[:, None, :], axis=-1)
    pos = vox.astype(jnp.float32) * _VS + _VMIN
    dvec = pos - means3d[:, None, :]
    mahal = jnp.einsum("nkd,nd,nkd->nk", dvec, inv_var, dvec)
    density = opacities[:, None] * jnp.exp(-0.5 * mahal)
    density = jnp.where(valid, density, 0.0)
    flat = vox[..., 0] * (_DIMS * _DIMS) + vox[..., 1] * _DIMS + vox[..., 2]
    flat = jnp.where(valid, flat, 0)
    grid_density = jnp.zeros((_DIMS ** 3,), jnp.float32).at[flat.reshape(-1)].add(density.reshape(-1))
    contrib = density[..., None] * features[:, None, :]
    grid_feats = jnp.zeros((_DIMS ** 3, n_dims), jnp.float32).at[flat.reshape(-1)].add(contrib.reshape(-1, n_dims))
    grid_feats = grid_feats.at[:, n_dims - 1].add(1e-05)

    blk = 8192
    out = pl.pallas_call(
        _norm_body,
        grid=(_DIMS ** 3 // blk,),
        in_specs=[
            pl.BlockSpec((blk, 1), lambda i: (i, 0)),
            pl.BlockSpec((blk, n_dims), lambda i: (i, 0)),
        ],
        out_specs=pl.BlockSpec((blk, n_dims + 1), lambda i: (i, 0)),
        out_shape=jax.ShapeDtypeStruct((_DIMS ** 3, n_dims + 1), jnp.float32),
    )(grid_density[:, None], grid_feats)
    return out.reshape(_DIMS, _DIMS, _DIMS, n_dims + 1)


# speedup vs baseline: 21.7661x; 15.3746x over previous
import jax
import jax.numpy as jnp
from jax import lax
from jax.experimental import pallas as pl
from jax.experimental.pallas import tpu as pltpu
from jax.experimental.pallas import tpu_sc as plsc

_N = 50000
_NFEAT = 8
_DIMS = 128
_VS = 2.0 / 128.0
_VMIN = -1.0
_K = 5
_EPS = 1e-06

_CH = 3584
_NP = 50176
_NCHUNK = _NP // _CH
_NPASS = 8
_SLABW = 64 * 128 * 9
_GB = 128
_MIDX = _CH + _GB


def _sc_body(g_hbm, i_hbm, sx_hbm, sy_hbm, grid_hbm,
             slab, sxc, syc, midx, growf, growi, mcv, sem):
    cid = lax.axis_index("c")
    sid = lax.axis_index("s")
    wid = sid * 2 + cid
    iota = lax.iota(jnp.int32, 16)
    zf16 = jnp.zeros((16,), jnp.float32)
    zi16 = jnp.zeros((16,), jnp.int32)

    def zmidx(i, carry):
        midx[pl.ds(i * 16, 16)] = zi16
        return carry
    lax.fori_loop(0, (_MIDX + 16) // 16, zmidx, 0)

    def do_pass(p, carry):
        slab_id = p * 32 + wid
        x_p = slab_id // 2
        yh0 = (slab_id % 2) * 64
        xf = x_p.astype(jnp.float32) * _VS + _VMIN

        def zslab(i, c):
            slab[pl.ds(i * 16, 16)] = zf16
            return c
        lax.fori_loop(0, _SLABW // 16, zslab, 0)

        def chunk(cc, c):
            pltpu.sync_copy(sx_hbm.at[pl.ds(cc * _CH, _CH)], sxc)
            pltpu.sync_copy(sy_hbm.at[pl.ds(cc * _CH, _CH)], syc)
            mcv[...] = zi16

            def scang(g, c1):
                sx = sxc[pl.ds(g * 16, 16)]
                sy = syc[pl.ds(g * 16, 16)]
                m = ((sx <= x_p) & (sx >= x_p - (_K - 1))
                     & (sy <= yh0 + 63) & (sy + (_K - 1) >= yh0))
                gidx = cc * _CH + g * 16 + iota
                key = jnp.where(m, iota, 16 + iota)
                _, gsort = plsc.sort_key_val(key, gidx)
                mcl0 = mcv[...][0]
                midx[pl.ds(mcl0, 16)] = gsort
                mcv[...] = mcv[...] + plsc.all_reduce_population_count(m)
                return c1
            lax.fori_loop(0, _CH // 16, scang, 0)

            def batch(b, c2):
                idxsl = midx.at[pl.ds(b * _GB, _GB)]
                pltpu.async_copy(g_hbm.at[idxsl], growf, sem).wait()
                pltpu.async_copy(i_hbm.at[idxsl], growi, sem).wait()
                mcb = mcv[...]

                def sub(j, c3):
                    rows = j * 16 + iota
                    act = (b * _GB + rows) < mcb

                    def gf(f):
                        return plsc.load_gather(
                            growf, [rows, jnp.full((16,), f, jnp.int32)])

                    def gi(f):
                        return plsc.load_gather(
                            growi, [rows, jnp.full((16,), f, jnp.int32)])

                    mx, my, mz = gf(0), gf(1), gf(2)
                    ivx, ivy, ivz = gf(3), gf(4), gf(5)
                    sxv, syv, szv = gi(0), gi(1), gi(2)
                    nxv, nyv, nzv = gi(3), gi(4), gi(5)

                    kx = x_p - sxv
                    dx = xf - mx
                    ex = jnp.exp(-0.5 * ivx * dx * dx)
                    ex = jnp.where((kx < nxv) & act, ex, 0.0)

                    syf = syv.astype(jnp.float32) * _VS + _VMIN - my
                    szf = szv.astype(jnp.float32) * _VS + _VMIN - mz
                    eys, ezs = [], []
                    for k in range(_K):
                        dy = syf + k * _VS
                        eys.append(jnp.where(
                            k < nyv, jnp.exp(-0.5 * ivy * dy * dy), 0.0))
                        dz = szf + k * _VS
                        ezs.append(jnp.where(
                            k < nzv, jnp.exp(-0.5 * ivz * dz * dz), 0.0))
                    ws = [gf(6 + c) for c in range(9)]
                    zbase = szv * 9
                    for ky in range(_K):
                        cy = syv + ky
                        inyh = (cy >= yh0) & (cy < yh0 + 64)
                        a = jnp.where(inyh, ex * eys[ky], 0.0)
                        pb = jnp.where(inyh, (cy - yh0) * 1152 + zbase, 0)
                        for kz in range(_K):
                            d = a * ezs[kz]
                            i0 = pb + kz * 9
                            for ch in range(9):
                                plsc.addupdate_scatter(
                                    slab, [i0 + ch], d * ws[ch])
                    return c3
                lax.fori_loop(0, _GB // 16, sub, 0)
                return c2
            mc = mcv[...][0]
            nb = (mc + _GB - 1) // _GB
            lax.fori_loop(0, nb, batch, 0)
            return c
        lax.fori_loop(0, _NCHUNK, chunk, 0)

        def norm(q, c):
            pidx = (q * 16 + iota) * 9
            dv = plsc.load_gather(slab, [pidx])
            r = 1.0 / jnp.maximum(dv, _EPS)
            for ch in range(1, 9):
                v = plsc.load_gather(slab, [pidx + ch])
                if ch == 8:
                    v = v + 1e-05
                plsc.store_scatter(slab, [pidx + ch], v * r)
            return c
        lax.fori_loop(0, _SLABW // 9 // 16, norm, 0)

        pltpu.sync_copy(slab, grid_hbm.at[slab_id])
        return carry
    lax.fori_loop(0, _NPASS, do_pass, 0)


def kernel(means3d, covs, opacities, features):
    n_dims = features.shape[1]
    var = jnp.diagonal(covs, axis1=-2, axis2=-1)
    ivar = 1.0 / var
    radii = 3.0 * jnp.sqrt(var)
    start = jnp.maximum(0, ((means3d - radii - _VMIN) / _VS).astype(jnp.int32))
    end = jnp.minimum(_DIMS, ((means3d + radii - _VMIN) / _VS).astype(jnp.int32) + 1)
    nv = jnp.clip(end - start, 0, _K)
    w = opacities[:, None] * jnp.concatenate(
        [jnp.ones_like(opacities)[:, None], features], axis=1)

    pad = _NP - _N
    gf = jnp.concatenate(
        [means3d, ivar, w, jnp.zeros((_N, 1), jnp.float32)], axis=1)
    gf = jnp.pad(gf, ((0, pad), (0, 0)))
    gi = jnp.concatenate([start, nv, jnp.zeros((_N, 2), jnp.int32)], axis=1)
    gi = jnp.pad(gi, ((0, pad), (0, 0)))
    sxa = jnp.pad(start[:, 0], (0, pad))
    sya = jnp.pad(start[:, 1], (0, pad))

    mesh = plsc.VectorSubcoreMesh(core_axis_name="c", subcore_axis_name="s")
    grid = pl.kernel(
        _sc_body,
        out_type=jax.ShapeDtypeStruct((256, _SLABW), jnp.float32),
        mesh=mesh,
        compiler_params=pltpu.CompilerParams(needs_layout_passes=False, use_tc_tiling_on_sc=False),
        scratch_types=[
            pltpu.VMEM((_SLABW,), jnp.float32),
            pltpu.VMEM((_CH,), jnp.int32),
            pltpu.VMEM((_CH,), jnp.int32),
            pltpu.VMEM((_MIDX + 16,), jnp.int32),
            pltpu.VMEM((_GB, 16), jnp.float32),
            pltpu.VMEM((_GB, 8), jnp.int32),
            pltpu.VMEM((16,), jnp.int32),
            pltpu.SemaphoreType.DMA,
        ],
    )(gf, gi, sxa, sya)
    return grid.reshape(_DIMS, 2, 64, _DIMS, n_dims + 1).reshape(
        _DIMS, _DIMS, _DIMS, n_dims + 1)


# speedup vs baseline: 30.8769x; 1.4186x over previous
import jax
import jax.numpy as jnp
from jax import lax
from jax.experimental import pallas as pl
from jax.experimental.pallas import tpu as pltpu
from jax.experimental.pallas import tpu_sc as plsc

_N = 50000
_NFEAT = 8
_DIMS = 128
_VS = 2.0 / 128.0
_VMIN = -1.0
_K = 5
_EPS = 1e-06

_CH = 3584
_NP = 50176
_NCHUNK = _NP // _CH
_NPASS = 8
_SLABW = 64 * 128 * 9
_GB = 128
_MIDX = _CH + _GB


def _sc_body(g_hbm, i_hbm, sxy_hbm, grid_hbm,
             slab, sxyc, midx, growf, growi, sem, sem2):
    cid = lax.axis_index("c")
    sid = lax.axis_index("s")
    wid = sid * 2 + cid
    iota = lax.iota(jnp.int32, 16)
    zf16 = jnp.zeros((16,), jnp.float32)
    zi16 = jnp.zeros((16,), jnp.int32)

    def zmidx(i, carry):
        midx[pl.ds(i * 16, 16)] = zi16
        return carry
    lax.fori_loop(0, (_MIDX + 16) // 16, zmidx, 0)

    def do_pass(p, carry):
        slab_id = p * 32 + wid
        x_p = slab_id // 2
        yh0 = (slab_id % 2) * 64
        xf = x_p.astype(jnp.float32) * _VS + _VMIN

        def zslab(i, c):
            slab[pl.ds(i * 16, 16)] = zf16
            return c
        lax.fori_loop(0, _SLABW // 16, zslab, 0)

        def chunk(cc, c):
            pltpu.sync_copy(sxy_hbm.at[:, pl.ds(cc * _CH, _CH)], sxyc)

            def scang(g, mc0):
                sx = sxyc[0, pl.ds(g * 16, 16)]
                sy = sxyc[1, pl.ds(g * 16, 16)]
                m = ((sx <= x_p) & (sx >= x_p - (_K - 1))
                     & (sy <= yh0 + 63) & (sy + (_K - 1) >= yh0))
                gidx = cc * _CH + g * 16 + iota
                incl = plsc.cumsum(m.astype(jnp.int32))
                off = jnp.where(m, mc0 + incl - 1, _MIDX + iota)
                plsc.store_scatter(midx, [off], gidx)
                return mc0 + plsc.all_reduce_population_count(m)[0]
            mc = lax.fori_loop(0, _CH // 16, scang, jnp.int32(0))

            def batch(b, c2):
                idxsl = midx.at[pl.ds(b * _GB, _GB)]
                ca = pltpu.async_copy(g_hbm.at[idxsl], growf, sem)
                cb = pltpu.async_copy(i_hbm.at[idxsl], growi, sem2)
                ca.wait()
                cb.wait()

                def sub(j, c3):
                    rows = j * 16 + iota
                    act = (b * _GB + rows) < mc

                    def gf(f):
                        return plsc.load_gather(
                            growf, [rows, jnp.full((16,), f, jnp.int32)])

                    def gi(f):
                        return plsc.load_gather(
                            growi, [rows, jnp.full((16,), f, jnp.int32)])

                    mx, my, mz = gf(0), gf(1), gf(2)
                    ivx, ivy, ivz = gf(3), gf(4), gf(5)
                    sxv, syv, szv = gi(0), gi(1), gi(2)
                    nxv, nyv, nzv = gi(3), gi(4), gi(5)

                    kx = x_p - sxv
                    dx = xf - mx
                    ex = jnp.exp(-0.5 * ivx * dx * dx)
                    ex = jnp.where((kx < nxv) & act, ex, 0.0)

                    syf = syv.astype(jnp.float32) * _VS + _VMIN - my
                    szf = szv.astype(jnp.float32) * _VS + _VMIN - mz
                    eys, ezs = [], []
                    for k in range(_K):
                        dy = syf + k * _VS
                        eys.append(jnp.where(
                            k < nyv, jnp.exp(-0.5 * ivy * dy * dy), 0.0))
                        dz = szf + k * _VS
                        ezs.append(jnp.where(
                            k < nzv, jnp.exp(-0.5 * ivz * dz * dz), 0.0))
                    ws = [gf(6 + c) for c in range(9)]
                    zbase = szv * 9
                    for ky in range(_K):
                        cy = syv + ky
                        inyh = (cy >= yh0) & (cy < yh0 + 64)
                        a = jnp.where(inyh, ex * eys[ky], 0.0)
                        pb = jnp.where(inyh, (cy - yh0) * 1152 + zbase, 0)
                        for kz in range(_K):
                            d = a * ezs[kz]
                            i0 = pb + kz * 9
                            for ch in range(9):
                                plsc.addupdate_scatter(
                                    slab, [i0 + ch], d * ws[ch])
                    return c3
                ns = jnp.minimum((mc - b * _GB + 15) // 16, _GB // 16)
                lax.fori_loop(0, ns, sub, 0)
                return c2
            nb = (mc + _GB - 1) // _GB
            lax.fori_loop(0, nb, batch, 0)
            return c
        lax.fori_loop(0, _NCHUNK, chunk, 0)

        def norm(q, c):
            pidx = (q * 16 + iota) * 9
            dv = plsc.load_gather(slab, [pidx])
            r = 1.0 / jnp.maximum(dv, _EPS)
            for ch in range(1, 9):
                v = plsc.load_gather(slab, [pidx + ch])
                if ch == 8:
                    v = v + 1e-05
                plsc.store_scatter(slab, [pidx + ch], v * r)
            return c
        lax.fori_loop(0, _SLABW // 9 // 16, norm, 0)

        pltpu.sync_copy(slab, grid_hbm.at[slab_id])
        return carry
    lax.fori_loop(0, _NPASS, do_pass, 0)


def kernel(means3d, covs, opacities, features):
    n_dims = features.shape[1]
    var = jnp.diagonal(covs, axis1=-2, axis2=-1)
    ivar = 1.0 / var
    radii = 3.0 * jnp.sqrt(var)
    start = jnp.maximum(0, ((means3d - radii - _VMIN) / _VS).astype(jnp.int32))
    end = jnp.minimum(_DIMS, ((means3d + radii - _VMIN) / _VS).astype(jnp.int32) + 1)
    nv = jnp.clip(end - start, 0, _K)
    w = opacities[:, None] * jnp.concatenate(
        [jnp.ones_like(opacities)[:, None], features], axis=1)

    pad = _NP - _N
    gf = jnp.concatenate(
        [means3d, ivar, w, jnp.zeros((_N, 1), jnp.float32)], axis=1)
    gf = jnp.pad(gf, ((0, pad), (0, 0)))
    gi = jnp.concatenate([start, nv, jnp.zeros((_N, 2), jnp.int32)], axis=1)
    gi = jnp.pad(gi, ((0, pad), (0, 0)))
    sxy = jnp.stack([jnp.pad(start[:, 0], (0, pad)),
                     jnp.pad(start[:, 1], (0, pad))], axis=0)

    mesh = plsc.VectorSubcoreMesh(core_axis_name="c", subcore_axis_name="s")
    grid = pl.kernel(
        _sc_body,
        out_type=jax.ShapeDtypeStruct((256, _SLABW), jnp.float32),
        mesh=mesh,
        compiler_params=pltpu.CompilerParams(needs_layout_passes=False, use_tc_tiling_on_sc=False),
        scratch_types=[
            pltpu.VMEM((_SLABW,), jnp.float32),
            pltpu.VMEM((2, _CH), jnp.int32),
            pltpu.VMEM((_MIDX + 16,), jnp.int32),
            pltpu.VMEM((_GB, 16), jnp.float32),
            pltpu.VMEM((_GB, 8), jnp.int32),
            pltpu.SemaphoreType.DMA,
            pltpu.SemaphoreType.DMA,
        ],
    )(gf, gi, sxy)
    return grid.reshape(_DIMS, 2, 64, _DIMS, n_dims + 1).reshape(
        _DIMS, _DIMS, _DIMS, n_dims + 1)


# speedup vs baseline: 36.0998x; 1.1692x over previous
import jax
import jax.numpy as jnp
from jax import lax
from jax.experimental import pallas as pl
from jax.experimental.pallas import tpu as pltpu
from jax.experimental.pallas import tpu_sc as plsc

_N = 50000
_NFEAT = 8
_DIMS = 128
_VS = 2.0 / 128.0
_VMIN = -1.0
_K = 5
_EPS = 1e-06

_NP = 50176
_BLK = 512
_NPASS = 8
_SLABW = 64 * 128 * 9
_GB = 128
_MIDX = _BLK + _GB


def _sc_body(g_hbm, i_hbm, sxy_hbm, cum_hbm, grid_hbm,
             slab, sxyb, midx, growf, growi, cumv, sem, sem2):
    cid = lax.axis_index("c")
    sid = lax.axis_index("s")
    wid = sid * 2 + cid
    iota = lax.iota(jnp.int32, 16)
    zf16 = jnp.zeros((16,), jnp.float32)
    zi16 = jnp.zeros((16,), jnp.int32)

    pltpu.sync_copy(cum_hbm, cumv)

    def zmidx(i, carry):
        midx[pl.ds(i * 16, 16)] = zi16
        return carry
    lax.fori_loop(0, (_MIDX + 16) // 16, zmidx, 0)

    def do_pass(p, carry):
        slab_id = p * 32 + wid
        x_p = slab_id // 2
        yh0 = (slab_id % 2) * 64
        xf = x_p.astype(jnp.float32) * _VS + _VMIN

        def zslab(i, c):
            slab[pl.ds(i * 16, 16)] = zf16
            return c
        lax.fori_loop(0, _SLABW // 16, zslab, 0)

        b0 = jnp.maximum(x_p - (_K - 1), 0)
        lo = cumv[pl.ds(b0, 16)][0]
        hi = cumv[pl.ds(x_p + 1, 16)][0]
        lo_al = (lo // 8) * 8
        cnt = hi - lo_al
        nblk = (cnt + _BLK - 1) // _BLK

        def blk(b, c):
            base = lo_al + b * _BLK
            pltpu.sync_copy(sxy_hbm.at[:, pl.ds(base, _BLK)], sxyb)

            def scang(g, mc0):
                sx = sxyb[0, pl.ds(g * 16, 16)]
                sy = sxyb[1, pl.ds(g * 16, 16)]
                m = ((sx <= x_p) & (sx >= x_p - (_K - 1))
                     & (sy <= yh0 + 63) & (sy + (_K - 1) >= yh0))
                gidx = base + g * 16 + iota
                incl = plsc.cumsum(m.astype(jnp.int32))
                off = jnp.where(m, mc0 + incl - 1, _MIDX + iota)
                plsc.store_scatter(midx, [off], gidx)
                return mc0 + plsc.all_reduce_population_count(m)[0]
            mcb = lax.fori_loop(0, _BLK // 16, scang, jnp.int32(0))

            def batch(bb, c2):
                idxsl = midx.at[pl.ds(bb * _GB, _GB)]
                ca = pltpu.async_copy(g_hbm.at[idxsl], growf, sem)
                cb = pltpu.async_copy(i_hbm.at[idxsl], growi, sem2)
                ca.wait()
                cb.wait()

                def sub(j, c3):
                    rows = j * 16 + iota
                    act = (bb * _GB + rows) < mcb

                    def gf(f):
                        return plsc.load_gather(
                            growf, [rows, jnp.full((16,), f, jnp.int32)])

                    def gi(f):
                        return plsc.load_gather(
                            growi, [rows, jnp.full((16,), f, jnp.int32)])

                    mx, my, mz = gf(0), gf(1), gf(2)
                    ivx, ivy, ivz = gf(3), gf(4), gf(5)
                    sxv, syv, szv = gi(0), gi(1), gi(2)
                    nxv, nyv, nzv = gi(3), gi(4), gi(5)

                    kx = x_p - sxv
                    dx = xf - mx
                    ex = jnp.exp(-0.5 * ivx * dx * dx)
                    ex = jnp.where((kx < nxv) & act, ex, 0.0)

                    syf = syv.astype(jnp.float32) * _VS + _VMIN - my
                    szf = szv.astype(jnp.float32) * _VS + _VMIN - mz
                    eys, ezs = [], []
                    for k in range(_K):
                        dy = syf + k * _VS
                        eys.append(jnp.where(
                            k < nyv, jnp.exp(-0.5 * ivy * dy * dy), 0.0))
                        dz = szf + k * _VS
                        ezs.append(jnp.where(
                            k < nzv, jnp.exp(-0.5 * ivz * dz * dz), 0.0))
                    ws = [gf(6 + c) for c in range(9)]
                    zbase = szv * 9
                    for ky in range(_K):
                        cy = syv + ky
                        inyh = (cy >= yh0) & (cy < yh0 + 64)
                        a = jnp.where(inyh, ex * eys[ky], 0.0)
                        pb = jnp.where(inyh, (cy - yh0) * 1152 + zbase, 0)
                        for kz in range(_K):
                            d = a * ezs[kz]
                            i0 = pb + kz * 9
                            for ch in range(9):
                                plsc.addupdate_scatter(
                                    slab, [i0 + ch], d * ws[ch])
                    return c3
                ns = jnp.minimum((mcb - bb * _GB + 15) // 16, _GB // 16)
                lax.fori_loop(0, ns, sub, 0)
                return c2
            nbb = (mcb + _GB - 1) // _GB
            lax.fori_loop(0, nbb, batch, 0)
            return c
        lax.fori_loop(0, nblk, blk, 0)

        def norm(q, c):
            pidx = (q * 16 + iota) * 9
            dv = plsc.load_gather(slab, [pidx])
            r = 1.0 / jnp.maximum(dv, _EPS)
            for ch in range(1, 9):
                v = plsc.load_gather(slab, [pidx + ch])
                if ch == 8:
                    v = v + 1e-05
                plsc.store_scatter(slab, [pidx + ch], v * r)
            return c
        lax.fori_loop(0, _SLABW // 9 // 16, norm, 0)

        pltpu.sync_copy(slab, grid_hbm.at[slab_id])
        return carry
    lax.fori_loop(0, _NPASS, do_pass, 0)


def kernel(means3d, covs, opacities, features):
    n_dims = features.shape[1]
    var = jnp.diagonal(covs, axis1=-2, axis2=-1)
    ivar = 1.0 / var
    radii = 3.0 * jnp.sqrt(var)
    start = jnp.maximum(0, ((means3d - radii - _VMIN) / _VS).astype(jnp.int32))
    end = jnp.minimum(_DIMS, ((means3d + radii - _VMIN) / _VS).astype(jnp.int32) + 1)
    nv = jnp.clip(end - start, 0, _K)
    w = opacities[:, None] * jnp.concatenate(
        [jnp.ones_like(opacities)[:, None], features], axis=1)

    order = jnp.argsort(start[:, 0])
    gf = jnp.concatenate(
        [means3d, ivar, w, jnp.zeros((_N, 1), jnp.float32)], axis=1)[order]
    gi = jnp.concatenate(
        [start, nv, jnp.zeros((_N, 2), jnp.int32)], axis=1)[order]
    sxs = start[order, 0]
    sys_ = start[order, 1]
    cum = jnp.searchsorted(sxs, jnp.arange(129, dtype=jnp.int32),
                           side="left").astype(jnp.int32)
    cum = jnp.pad(cum, (0, 160 - 129), mode="edge")

    pad = _NP - _N
    gf = jnp.pad(gf, ((0, pad), (0, 0)))
    gi = jnp.pad(gi, ((0, pad), (0, 0)))
    sxy = jnp.stack([jnp.pad(sxs, (0, pad + _BLK), constant_values=10000),
                     jnp.pad(sys_, (0, pad + _BLK))], axis=0)

    mesh = plsc.VectorSubcoreMesh(core_axis_name="c", subcore_axis_name="s")
    grid = pl.kernel(
        _sc_body,
        out_type=jax.ShapeDtypeStruct((256, _SLABW), jnp.float32),
        mesh=mesh,
        compiler_params=pltpu.CompilerParams(
            needs_layout_passes=False, use_tc_tiling_on_sc=False),
        scratch_types=[
            pltpu.VMEM((_SLABW,), jnp.float32),
            pltpu.VMEM((2, _BLK), jnp.int32),
            pltpu.VMEM((_MIDX + 16,), jnp.int32),
            pltpu.VMEM((_GB, 16), jnp.float32),
            pltpu.VMEM((_GB, 8), jnp.int32),
            pltpu.VMEM((160,), jnp.int32),
            pltpu.SemaphoreType.DMA,
            pltpu.SemaphoreType.DMA,
        ],
    )(gf, gi, sxy, cum)
    return grid.reshape(_DIMS, 2, 64, _DIMS, n_dims + 1).reshape(
        _DIMS, _DIMS, _DIMS, n_dims + 1)


# speedup vs baseline: 40.8150x; 1.1306x over previous
import jax
import jax.numpy as jnp
from jax import lax
from jax.experimental import pallas as pl
from jax.experimental.pallas import tpu as pltpu
from jax.experimental.pallas import tpu_sc as plsc

_N = 50000
_NFEAT = 8
_DIMS = 128
_VS = 2.0 / 128.0
_VMIN = -1.0
_K = 5
_EPS = 1e-06

_NP = 50176
_BLK = 1024
_NPASS = 8
_SLABW = 64 * 128 * 9
_GB = 128
_MIDX = _BLK + _GB
_NBB = _MIDX // _GB


def _sc_body(g_hbm, i_hbm, sxy_hbm, cum_hbm, zeros_hbm, grid_hbm,
             slab, sxyb, midx, growf, growi, cumv, sem, sem2):
    cid = lax.axis_index("c")
    sid = lax.axis_index("s")
    wid = sid * 2 + cid
    iota = lax.iota(jnp.int32, 16)
    zf16 = jnp.zeros((16,), jnp.float32)
    zi16 = jnp.zeros((16,), jnp.int32)

    pltpu.sync_copy(cum_hbm, cumv)

    def zmidx(i, carry):
        midx[pl.ds(i * 16, 16)] = zi16
        return carry
    lax.fori_loop(0, (_MIDX + 16) // 16, zmidx, 0)

    def do_pass(p, carry):
        slab_id = p * 32 + wid
        x_p = slab_id // 2
        yh0 = (slab_id % 2) * 64
        xf = x_p.astype(jnp.float32) * _VS + _VMIN

        pltpu.sync_copy(zeros_hbm, slab)

        b0 = jnp.maximum(x_p - (_K - 1), 0)
        lo = cumv[pl.ds(b0, 16)][0]
        hi = cumv[pl.ds(x_p + 1, 16)][0]
        lo_al = (lo // 8) * 8
        cnt = hi - lo_al
        nblk = (cnt + _BLK - 1) // _BLK

        def blk(b, c):
            base = lo_al + b * _BLK
            pltpu.sync_copy(sxy_hbm.at[:, pl.ds(base, _BLK)], sxyb)

            def scang(g, mc0):
                sx = sxyb[0, pl.ds(g * 16, 16)]
                sy = sxyb[1, pl.ds(g * 16, 16)]
                m = ((sx <= x_p) & (sx >= x_p - (_K - 1))
                     & (sy <= yh0 + 63) & (sy + (_K - 1) >= yh0))
                gidx = base + g * 16 + iota
                incl = plsc.cumsum(m.astype(jnp.int32))
                off = jnp.where(m, mc0 + incl - 1, _MIDX + iota)
                plsc.store_scatter(midx, [off], gidx)
                return mc0 + plsc.all_reduce_population_count(m)[0]
            mcb = lax.fori_loop(0, _BLK // 16, scang, jnp.int32(0))

            nbb = (mcb + _GB - 1) // _GB

            def fire(bb, c1):
                idxsl = midx.at[pl.ds(bb * _GB, _GB)]
                pltpu.async_copy(g_hbm.at[idxsl], growf.at[bb], sem)
                pltpu.async_copy(i_hbm.at[idxsl], growi.at[bb], sem2)
                return c1
            lax.fori_loop(0, nbb, fire, 0)

            def drain(bb, c1):
                pltpu.make_async_copy(
                    g_hbm.at[pl.ds(0, _GB)], growf.at[0], sem).wait()
                pltpu.make_async_copy(
                    i_hbm.at[pl.ds(0, _GB)], growi.at[0], sem2).wait()
                return c1
            lax.fori_loop(0, nbb, drain, 0)

            def batch(bb, c2):
                bbv = jnp.full((16,), bb, jnp.int32)

                def sub(j, c3):
                    rows = j * 16 + iota
                    act = (bb * _GB + rows) < mcb

                    def gf(f):
                        return plsc.load_gather(
                            growf, [bbv, rows, jnp.full((16,), f, jnp.int32)])

                    def gi(f):
                        return plsc.load_gather(
                            growi, [bbv, rows, jnp.full((16,), f, jnp.int32)])

                    mx, my, mz = gf(0), gf(1), gf(2)
                    ivx, ivy, ivz = gf(3), gf(4), gf(5)
                    sxv, syv, szv = gi(0), gi(1), gi(2)
                    nxv, nyv, nzv = gi(3), gi(4), gi(5)

                    kx = x_p - sxv
                    dx = xf - mx
                    ex = jnp.exp(-0.5 * ivx * dx * dx)
                    ex = jnp.where((kx < nxv) & act, ex, 0.0)

                    syf = syv.astype(jnp.float32) * _VS + _VMIN - my
                    szf = szv.astype(jnp.float32) * _VS + _VMIN - mz
                    eys, ezs = [], []
                    for k in range(_K):
                        dy = syf + k * _VS
                        eys.append(jnp.where(
                            k < nyv, jnp.exp(-0.5 * ivy * dy * dy), 0.0))
                        dz = szf + k * _VS
                        ezs.append(jnp.where(
                            k < nzv, jnp.exp(-0.5 * ivz * dz * dz), 0.0))
                    ws = [gf(6 + c) for c in range(9)]
                    zbase = szv * 9
                    for ky in range(_K):
                        cy = syv + ky
                        inyh = (cy >= yh0) & (cy < yh0 + 64)
                        a = jnp.where(inyh, ex * eys[ky], 0.0)
                        pb = jnp.where(inyh, (cy - yh0) * 1152 + zbase, 0)
                        for kz in range(_K):
                            d = a * ezs[kz]
                            i0 = pb + kz * 9
                            for ch in range(9):
                                plsc.addupdate_scatter(
                                    slab, [i0 + ch], d * ws[ch])
                    return c3
                ns = jnp.minimum((mcb - bb * _GB + 15) // 16, _GB // 16)
                lax.fori_loop(0, ns, sub, 0)
                return c2
            lax.fori_loop(0, nbb, batch, 0)
            return c
        lax.fori_loop(0, nblk, blk, 0)

        def norm(q, c):
            pidx = (q * 16 + iota) * 9
            dv = plsc.load_gather(slab, [pidx])
            r = 1.0 / jnp.maximum(dv, _EPS)
            for ch in range(1, 9):
                v = plsc.load_gather(slab, [pidx + ch])
                if ch == 8:
                    v = v + 1e-05
                plsc.store_scatter(slab, [pidx + ch], v * r)
            return c
        lax.fori_loop(0, _SLABW // 9 // 16, norm, 0)

        pltpu.sync_copy(slab, grid_hbm.at[slab_id])
        return carry
    lax.fori_loop(0, _NPASS, do_pass, 0)


def kernel(means3d, covs, opacities, features):
    n_dims = features.shape[1]
    var = jnp.diagonal(covs, axis1=-2, axis2=-1)
    ivar = 1.0 / var
    radii = 3.0 * jnp.sqrt(var)
    start = jnp.maximum(0, ((means3d - radii - _VMIN) / _VS).astype(jnp.int32))
    end = jnp.minimum(_DIMS, ((means3d + radii - _VMIN) / _VS).astype(jnp.int32) + 1)
    nv = jnp.clip(end - start, 0, _K)
    w = opacities[:, None] * jnp.concatenate(
        [jnp.ones_like(opacities)[:, None], features], axis=1)

    order = jnp.argsort(start[:, 0])
    gf = jnp.concatenate(
        [means3d, ivar, w, jnp.zeros((_N, 1), jnp.float32)], axis=1)[order]
    gi = jnp.concatenate(
        [start, nv, jnp.zeros((_N, 2), jnp.int32)], axis=1)[order]
    sxs = start[order, 0]
    sys_ = start[order, 1]
    cum = jnp.searchsorted(sxs, jnp.arange(129, dtype=jnp.int32),
                           side="left").astype(jnp.int32)
    cum = jnp.pad(cum, (0, 160 - 129), mode="edge")

    pad = _NP - _N
    gf = jnp.pad(gf, ((0, pad), (0, 0)))
    gi = jnp.pad(gi, ((0, pad), (0, 0)))
    sxy = jnp.stack([jnp.pad(sxs, (0, pad + _BLK), constant_values=10000),
                     jnp.pad(sys_, (0, pad + _BLK))], axis=0)

    mesh = plsc.VectorSubcoreMesh(core_axis_name="c", subcore_axis_name="s")
    grid = pl.kernel(
        _sc_body,
        out_type=jax.ShapeDtypeStruct((256, _SLABW), jnp.float32),
        mesh=mesh,
        compiler_params=pltpu.CompilerParams(
            needs_layout_passes=False, use_tc_tiling_on_sc=False),
        scratch_types=[
            pltpu.VMEM((_SLABW,), jnp.float32),
            pltpu.VMEM((2, _BLK), jnp.int32),
            pltpu.VMEM((_MIDX + 16,), jnp.int32),
            pltpu.VMEM((_NBB, _GB, 16), jnp.float32),
            pltpu.VMEM((_NBB, _GB, 8), jnp.int32),
            pltpu.VMEM((160,), jnp.int32),
            pltpu.SemaphoreType.DMA,
            pltpu.SemaphoreType.DMA,
        ],
    )(gf, gi, sxy, cum, jnp.zeros((_SLABW,), jnp.float32))
    return grid.reshape(_DIMS, 2, 64, _DIMS, n_dims + 1).reshape(
        _DIMS, _DIMS, _DIMS, n_dims + 1)
